# baseline (speedup 1.0000x reference)
import functools

import jax
import jax.numpy as jnp
from jax import lax
from jax.experimental import pallas as pl
from jax.experimental.pallas import tpu as pltpu
from jax.experimental.pallas import tpu_sc as plsc

EMBED = 1024
FF = 1024
NEXP = 64
CAP = 128
TOKENS = 8192
CHUNK = 512
NCHUNK = TOKENS // CHUNK
OVERFLOW = NEXP * CAP
EPG = 2
ROWS_PAD = OVERFLOW + EPG * CAP

NC, NS = 2, 16
NW = NC * NS
TPW = TOKENS // NW
JCH = 8
JW = TPW // JCH


def _router_body(x_ref, w_ref, b_ref, slots_ref, loss_ref, counts_ref, imp_ref):
    i = pl.program_id(0)

    @pl.when(i == 0)
    def _init():
        counts_ref[...] = jnp.zeros_like(counts_ref)
        imp_ref[...] = jnp.zeros_like(imp_ref)

    x = x_ref[...]
    w = w_ref[...]
    logits = lax.dot_general(x, w, (((1,), (1,)), ((), ())),
                             preferred_element_type=jnp.float32)
    logits = logits + b_ref[...]

    m = jnp.max(logits, axis=1, keepdims=True)
    p = jnp.exp(logits - m)
    scores = p / jnp.sum(p, axis=1, keepdims=True)
    imp_ref[...] += jnp.sum(scores, axis=0, keepdims=True)

    col = lax.broadcasted_iota(jnp.int32, (CHUNK, NEXP), 1)
    amax = jnp.min(jnp.where(logits == m, col, NEXP), axis=1)
    oh = (col == amax[:, None]).astype(jnp.float32)

    r_i = lax.broadcasted_iota(jnp.int32, (CHUNK, CHUNK), 0)
    c_i = lax.broadcasted_iota(jnp.int32, (CHUNK, CHUNK), 1)
    ltri = (r_i > c_i).astype(jnp.float32)
    within = lax.dot_general(ltri, oh, (((1,), (0,)), ((), ())),
                             preferred_element_type=jnp.float32)
    rank_f = (jnp.sum(within * oh, axis=1)
              + jnp.sum(oh * counts_ref[0, :][None, :], axis=1))
    counts_ref[...] += jnp.sum(oh, axis=0, keepdims=True)

    rank = rank_f.astype(jnp.int32)
    slot = jnp.where(rank < CAP, amax * CAP + rank, OVERFLOW)
    slots_ref[0, 0, :] = slot

    @pl.when(i == NCHUNK - 1)
    def _fin():
        counts = counts_ref[0, :]
        imp = imp_ref[0, :]
        loss = (NEXP / (float(TOKENS) * float(TOKENS))
                * jnp.sum(counts * imp))
        loss_ref[...] = jnp.broadcast_to(loss, loss_ref.shape)


_router_call = pl.pallas_call(
    _router_body,
    grid=(NCHUNK,),
    in_specs=[
        pl.BlockSpec((CHUNK, EMBED), lambda i: (i, 0)),
        pl.BlockSpec((NEXP, EMBED), lambda i: (0, 0)),
        pl.BlockSpec((1, NEXP), lambda i: (0, 0)),
    ],
    out_specs=[
        pl.BlockSpec((1, 1, CHUNK), lambda i: (i, 0, 0)),
        pl.BlockSpec((1, 128), lambda i: (0, 0)),
    ],
    out_shape=[
        jax.ShapeDtypeStruct((NCHUNK, 1, CHUNK), jnp.int32),
        jax.ShapeDtypeStruct((1, 128), jnp.float32),
    ],
    scratch_shapes=[
        pltpu.VMEM((1, NEXP), jnp.float32),
        pltpu.VMEM((1, NEXP), jnp.float32),
    ],
)


HGRP = NEXP // (2 * EPG)
HROWS = NEXP * CAP // 2


def _ffn_compute(x_ref, g_ref, e_ref, d_ref, out_ref):
    for k in range(EPG):
        x = x_ref[pl.ds(k * CAP, CAP), :].astype(jnp.bfloat16)
        g = g_ref[k].astype(jnp.bfloat16)
        en = e_ref[k].astype(jnp.bfloat16)
        de = d_ref[k].astype(jnp.bfloat16)
        h1 = lax.dot_general(x, g, (((1,), (1,)), ((), ())),
                             preferred_element_type=jnp.float32)
        h2 = lax.dot_general(x, en, (((1,), (1,)), ((), ())),
                             preferred_element_type=jnp.float32)
        act = (h1 * jax.nn.sigmoid(h1) * h2).astype(jnp.bfloat16)
        out_ref[pl.ds(k * CAP, CAP), :] = lax.dot_general(
            act, de, (((1,), (1,)), ((), ())),
            preferred_element_type=jnp.float32)


def _ffn_a_body(x_ref, g_ref, e_ref, d_ref, out_ref):
    _ffn_compute(x_ref, g_ref, e_ref, d_ref, out_ref)


def _ffn_b_body(x_ref, g_ref, e_ref, d_ref, alias_ref, out_ref):
    i = pl.program_id(0)

    @pl.when(i < HGRP)
    def _compute():
        _ffn_compute(x_ref, g_ref, e_ref, d_ref, out_ref)

    @pl.when(i == HGRP)
    def _zero():
        out_ref[...] = jnp.zeros_like(out_ref)


_ffn_a = pl.pallas_call(
    _ffn_a_body,
    grid=(HGRP,),
    in_specs=[
        pl.BlockSpec((EPG * CAP, EMBED), lambda i: (i, 0)),
        pl.BlockSpec((EPG, FF, EMBED), lambda i: (i, 0, 0)),
        pl.BlockSpec((EPG, FF, EMBED), lambda i: (i, 0, 0)),
        pl.BlockSpec((EPG, EMBED, FF), lambda i: (i, 0, 0)),
    ],
    out_specs=pl.BlockSpec((EPG * CAP, EMBED), lambda i: (i, 0)),
    out_shape=jax.ShapeDtypeStruct((ROWS_PAD, EMBED), jnp.float32),
)

_ffn_b = pl.pallas_call(
    _ffn_b_body,
    grid=(HGRP + 1,),
    in_specs=[
        pl.BlockSpec((EPG * CAP, EMBED), lambda i: (jnp.minimum(i, HGRP - 1), 0)),
        pl.BlockSpec((EPG, FF, EMBED),
                     lambda i: (jnp.minimum(i, HGRP - 1) + HGRP, 0, 0)),
        pl.BlockSpec((EPG, FF, EMBED),
                     lambda i: (jnp.minimum(i, HGRP - 1) + HGRP, 0, 0)),
        pl.BlockSpec((EPG, EMBED, FF),
                     lambda i: (jnp.minimum(i, HGRP - 1) + HGRP, 0, 0)),
        pl.BlockSpec(memory_space=pl.ANY),
    ],
    out_specs=pl.BlockSpec((EPG * CAP, EMBED), lambda i: (i + HGRP, 0)),
    out_shape=jax.ShapeDtypeStruct((ROWS_PAD, EMBED), jnp.float32),
    input_output_aliases={4: 0},
)


@functools.cache
def _sc_kernels():
    mesh = plsc.VectorSubcoreMesh(core_axis_name="c", subcore_axis_name="s",
                                  num_cores=NC, num_subcores=NS)
    scratch = [
        pltpu.VMEM((JCH, JW), jnp.int32),
        pltpu.VMEM((JW, EMBED), jnp.float32),
        pltpu.VMEM((JW, EMBED), jnp.float32),
        pltpu.SemaphoreType.DMA,
        pltpu.SemaphoreType.DMA,
        pltpu.SemaphoreType.DMA,
        pltpu.SemaphoreType.DMA,
    ]

    def make_dispatch(lo):
        @functools.partial(
            pl.kernel,
            out_type=jax.ShapeDtypeStruct((HROWS, EMBED), jnp.float32),
            mesh=mesh,
            scratch_types=[
                pltpu.VMEM((TPW + 16,), jnp.int32),
                pltpu.VMEM((TPW + 32,), jnp.int32),
                pltpu.VMEM((TPW + 32,), jnp.int32),
                pltpu.VMEM((TPW // 32, 32), jnp.int32),
                pltpu.VMEM((TPW // 32, 32), jnp.int32),
                pltpu.VMEM((32, EMBED), jnp.float32),
                pltpu.VMEM((32, EMBED), jnp.float32),
                pltpu.SemaphoreType.DMA,
                pltpu.SemaphoreType.DMA,
            ],
        )
        def dispatch_sc(x_hbm, slots_hbm, xd_hbm, slot_v, tokf, slotf,
                        tok2d, slot2d, buf0, buf1, gsem, ssem):
            wid = lax.axis_index("s") * NC + lax.axis_index("c")
            pltpu.sync_copy(slots_hbm.at[wid], slot_v.at[pl.ds(0, TPW)])
            cnt = jnp.int32(0)

            base = wid * TPW

            def _scan(i, carry):
                c, ltk, lsl = carry
                s = slot_v[pl.ds(i, 16)][0]
                d = s - lo
                ing = (d >= 0) & (d < HROWS)
                tok = base + i

                @pl.when(ing)
                def _app():
                    tokf[pl.ds(c, 16)] = jnp.full((16,), tok, jnp.int32)
                    slotf[pl.ds(c, 16)] = jnp.full((16,), d, jnp.int32)

                one = jnp.where(ing, jnp.int32(1), jnp.int32(0))
                return (c + one, jnp.where(ing, tok, ltk),
                        jnp.where(ing, d, lsl))

            cnt, ltk, lsl = lax.fori_loop(
                0, TPW, _scan, (cnt, jnp.int32(0), jnp.int32(0)))

            @pl.when(cnt > 0)
            def _pad():
                tokf[pl.ds(cnt, 16)] = jnp.full((16,), ltk, jnp.int32)
                tokf[pl.ds(cnt + 16, 16)] = jnp.full((16,), ltk, jnp.int32)
                slotf[pl.ds(cnt, 16)] = jnp.full((16,), lsl, jnp.int32)
                slotf[pl.ds(cnt + 16, 16)] = jnp.full((16,), lsl, jnp.int32)

            for q in range(TPW // 32):
                tok2d[q, pl.ds(0, 16)] = tokf[pl.ds(q * 32, 16)]
                tok2d[q, pl.ds(16, 16)] = tokf[pl.ds(q * 32 + 16, 16)]
                slot2d[q, pl.ds(0, 16)] = slotf[pl.ds(q * 32, 16)]
                slot2d[q, pl.ds(16, 16)] = slotf[pl.ds(q * 32 + 16, 16)]

            for j in range(TPW // 32):
                @pl.when(j * 32 < cnt)
                def _move(j=j):
                    buf = buf0 if j % 2 == 0 else buf1
                    pltpu.async_copy(x_hbm.at[tok2d.at[j]], buf, gsem).wait()
                    pltpu.async_copy(buf, xd_hbm.at[slot2d.at[j]], ssem).wait()

        return dispatch_sc

    dispatch_a = make_dispatch(0)
    dispatch_b = make_dispatch(HROWS)

    @functools.partial(
        pl.kernel,
        out_type=jax.ShapeDtypeStruct((TOKENS, EMBED), jnp.float32),
        mesh=mesh, scratch_types=scratch,
    )
    def combine_sc(out2_hbm, slots_hbm, out_hbm, idx_v, rows0, rows1,
                   gsem0, gsem1, ssem0, ssem1):
        wid = lax.axis_index("s") * NC + lax.axis_index("c")
        bufs, gsems, ssems = (rows0, rows1), (gsem0, gsem1), (ssem0, ssem1)
        pltpu.sync_copy(slots_hbm.at[wid], idx_v)
        gets = [None] * JCH
        puts = [None] * JCH
        gets[0] = pltpu.async_copy(
            out2_hbm.at[idx_v.at[0]], bufs[0], gsems[0])
        for j in range(JCH):
            p = j % 2
            gets[j].wait()
            puts[j] = pltpu.async_copy(
                bufs[p], out_hbm.at[pl.ds(wid * TPW + j * JW, JW)], ssems[p])
            if j + 1 < JCH:
                if j - 1 >= 0:
                    puts[j - 1].wait()
                gets[j + 1] = pltpu.async_copy(
                    out2_hbm.at[idx_v.at[j + 1]],
                    bufs[(j + 1) % 2], gsems[(j + 1) % 2])
        puts[JCH - 2].wait()
        puts[JCH - 1].wait()

    return dispatch_a, dispatch_b, combine_sc


def kernel(x_ble, W_router, b_router, gate_nfe, enc_nfe, dec_nef):
    b, l, e = x_ble.shape
    x_te = x_ble.reshape(b * l, e)
    slots16, loss = _router_call(x_te, W_router, b_router.reshape(1, NEXP))
    slots2 = slots16.reshape(NW, TPW)
    slots3 = slots16.reshape(NW, JCH, JW)
    dispatch_a, dispatch_b, combine_sc = _sc_kernels()
    xd_a = dispatch_a(x_te, slots2)
    xd_b = dispatch_b(x_te, slots2)
    o2_part = _ffn_a(xd_a, gate_nfe, enc_nfe, dec_nef)
    out2 = _ffn_b(xd_b, gate_nfe, enc_nfe, dec_nef, o2_part)
    out_te = combine_sc(out2, slots3)
    return out_te.reshape(b, l, e), loss[0, 0]


# speedup vs baseline: 1.0190x; 1.0190x over previous
import functools

import jax
import jax.numpy as jnp
from jax import lax
from jax.experimental import pallas as pl
from jax.experimental.pallas import tpu as pltpu
from jax.experimental.pallas import tpu_sc as plsc

EMBED = 1024
FF = 1024
NEXP = 64
CAP = 128
TOKENS = 8192
CHUNK = 512
NCHUNK = TOKENS // CHUNK
OVERFLOW = NEXP * CAP
EPG = 2
ROWS_PAD = OVERFLOW + EPG * CAP

NC, NS = 2, 16
NW = NC * NS
TPW = TOKENS // NW
JCH = 8
JW = TPW // JCH


def _router_body(x_ref, w_ref, b_ref, slots_ref, loss_ref, counts_ref, imp_ref):
    i = pl.program_id(0)

    @pl.when(i == 0)
    def _init():
        counts_ref[...] = jnp.zeros_like(counts_ref)
        imp_ref[...] = jnp.zeros_like(imp_ref)

    x = x_ref[...]
    w = w_ref[...]
    logits = lax.dot_general(x, w, (((1,), (1,)), ((), ())),
                             preferred_element_type=jnp.float32)
    logits = logits + b_ref[...]

    m = jnp.max(logits, axis=1, keepdims=True)
    p = jnp.exp(logits - m)
    scores = p / jnp.sum(p, axis=1, keepdims=True)
    imp_ref[...] += jnp.sum(scores, axis=0, keepdims=True)

    col = lax.broadcasted_iota(jnp.int32, (CHUNK, NEXP), 1)
    amax = jnp.min(jnp.where(logits == m, col, NEXP), axis=1)
    oh = (col == amax[:, None]).astype(jnp.float32)

    r_i = lax.broadcasted_iota(jnp.int32, (CHUNK, CHUNK), 0)
    c_i = lax.broadcasted_iota(jnp.int32, (CHUNK, CHUNK), 1)
    ltri = (r_i > c_i).astype(jnp.float32)
    within = lax.dot_general(ltri, oh, (((1,), (0,)), ((), ())),
                             preferred_element_type=jnp.float32)
    rank_f = (jnp.sum(within * oh, axis=1)
              + jnp.sum(oh * counts_ref[0, :][None, :], axis=1))
    counts_ref[...] += jnp.sum(oh, axis=0, keepdims=True)

    rank = rank_f.astype(jnp.int32)
    slot = jnp.where(rank < CAP, amax * CAP + rank, OVERFLOW)
    slots_ref[0, 0, :] = slot

    @pl.when(i == NCHUNK - 1)
    def _fin():
        counts = counts_ref[0, :]
        imp = imp_ref[0, :]
        loss = (NEXP / (float(TOKENS) * float(TOKENS))
                * jnp.sum(counts * imp))
        loss_ref[...] = jnp.broadcast_to(loss, loss_ref.shape)


_router_call = pl.pallas_call(
    _router_body,
    grid=(NCHUNK,),
    in_specs=[
        pl.BlockSpec((CHUNK, EMBED), lambda i: (i, 0)),
        pl.BlockSpec((NEXP, EMBED), lambda i: (0, 0)),
        pl.BlockSpec((1, NEXP), lambda i: (0, 0)),
    ],
    out_specs=[
        pl.BlockSpec((1, 1, CHUNK), lambda i: (i, 0, 0)),
        pl.BlockSpec((1, 128), lambda i: (0, 0)),
    ],
    out_shape=[
        jax.ShapeDtypeStruct((NCHUNK, 1, CHUNK), jnp.int32),
        jax.ShapeDtypeStruct((1, 128), jnp.float32),
    ],
    scratch_shapes=[
        pltpu.VMEM((1, NEXP), jnp.float32),
        pltpu.VMEM((1, NEXP), jnp.float32),
    ],
)


def _ffn_body(x_ref, g_ref, e_ref, d_ref, out_ref):
    i = pl.program_id(0)

    @pl.when(i < NEXP // EPG)
    def _compute():
        for k in range(EPG):
            x = x_ref[pl.ds(k * CAP, CAP), :].astype(jnp.bfloat16)
            g = g_ref[k].astype(jnp.bfloat16)
            en = e_ref[k].astype(jnp.bfloat16)
            de = d_ref[k].astype(jnp.bfloat16)
            h1 = lax.dot_general(x, g, (((1,), (1,)), ((), ())),
                                 preferred_element_type=jnp.float32)
            h2 = lax.dot_general(x, en, (((1,), (1,)), ((), ())),
                                 preferred_element_type=jnp.float32)
            act = (h1 * jax.nn.sigmoid(h1) * h2).astype(jnp.bfloat16)
            out_ref[pl.ds(k * CAP, CAP), :] = lax.dot_general(
                act, de, (((1,), (1,)), ((), ())),
                preferred_element_type=jnp.float32)

    @pl.when(i == NEXP // EPG)
    def _zero():
        out_ref[...] = jnp.zeros_like(out_ref)


_ffn_call = pl.pallas_call(
    _ffn_body,
    grid=(NEXP // EPG + 1,),
    in_specs=[
        pl.BlockSpec((EPG * CAP, EMBED), lambda i: (i, 0)),
        pl.BlockSpec((EPG, FF, EMBED),
                     lambda i: (jnp.minimum(i, NEXP // EPG - 1), 0, 0)),
        pl.BlockSpec((EPG, FF, EMBED),
                     lambda i: (jnp.minimum(i, NEXP // EPG - 1), 0, 0)),
        pl.BlockSpec((EPG, EMBED, FF),
                     lambda i: (jnp.minimum(i, NEXP // EPG - 1), 0, 0)),
    ],
    out_specs=pl.BlockSpec((EPG * CAP, EMBED), lambda i: (i, 0)),
    out_shape=jax.ShapeDtypeStruct((ROWS_PAD, EMBED), jnp.float32),
)


@functools.cache
def _sc_kernels():
    mesh = plsc.VectorSubcoreMesh(core_axis_name="c", subcore_axis_name="s",
                                  num_cores=NC, num_subcores=NS)
    scratch = [
        pltpu.VMEM((JCH, JW), jnp.int32),
        pltpu.VMEM((JW, EMBED), jnp.float32),
        pltpu.VMEM((JW, EMBED), jnp.float32),
        pltpu.SemaphoreType.DMA,
        pltpu.SemaphoreType.DMA,
        pltpu.SemaphoreType.DMA,
        pltpu.SemaphoreType.DMA,
    ]


    @functools.partial(
        pl.kernel,
        out_type=jax.ShapeDtypeStruct((ROWS_PAD, EMBED), jnp.float32),
        mesh=mesh, scratch_types=scratch,
    )
    def dispatch_sc(x_hbm, slots_hbm, xdisp_hbm, idx_v, rows0, rows1,
                    gsem0, gsem1, ssem0, ssem1):
        wid = lax.axis_index("s") * NC + lax.axis_index("c")
        bufs, gsems, ssems = (rows0, rows1), (gsem0, gsem1), (ssem0, ssem1)
        pltpu.sync_copy(slots_hbm.at[wid], idx_v)
        gets = [None] * JCH
        puts = [None] * JCH
        gets[0] = pltpu.async_copy(
            x_hbm.at[pl.ds(wid * TPW, JW)], bufs[0], gsems[0])
        for j in range(JCH):
            p = j % 2
            gets[j].wait()
            puts[j] = pltpu.async_copy(
                bufs[p], xdisp_hbm.at[idx_v.at[j]], ssems[p])
            if j + 1 < JCH:
                if j - 1 >= 0:
                    puts[j - 1].wait()
                gets[j + 1] = pltpu.async_copy(
                    x_hbm.at[pl.ds(wid * TPW + (j + 1) * JW, JW)],
                    bufs[(j + 1) % 2], gsems[(j + 1) % 2])
        puts[JCH - 2].wait()
        puts[JCH - 1].wait()

    @functools.partial(
        pl.kernel,
        out_type=jax.ShapeDtypeStruct((TOKENS, EMBED), jnp.float32),
        mesh=mesh, scratch_types=scratch,
    )
    def combine_sc(out2_hbm, slots_hbm, out_hbm, idx_v, rows0, rows1,
                   gsem0, gsem1, ssem0, ssem1):
        wid = lax.axis_index("s") * NC + lax.axis_index("c")
        bufs, gsems, ssems = (rows0, rows1), (gsem0, gsem1), (ssem0, ssem1)
        pltpu.sync_copy(slots_hbm.at[wid], idx_v)
        gets = [None] * JCH
        puts = [None] * JCH
        gets[0] = pltpu.async_copy(
            out2_hbm.at[idx_v.at[0]], bufs[0], gsems[0])
        for j in range(JCH):
            p = j % 2
            gets[j].wait()
            puts[j] = pltpu.async_copy(
                bufs[p], out_hbm.at[pl.ds(wid * TPW + j * JW, JW)], ssems[p])
            if j + 1 < JCH:
                if j - 1 >= 0:
                    puts[j - 1].wait()
                gets[j + 1] = pltpu.async_copy(
                    out2_hbm.at[idx_v.at[j + 1]],
                    bufs[(j + 1) % 2], gsems[(j + 1) % 2])
        puts[JCH - 2].wait()
        puts[JCH - 1].wait()

    return dispatch_sc, combine_sc


def kernel(x_ble, W_router, b_router, gate_nfe, enc_nfe, dec_nef):
    b, l, e = x_ble.shape
    x_te = x_ble.reshape(b * l, e)
    slots16, loss = _router_call(x_te, W_router, b_router.reshape(1, NEXP))
    slots3 = slots16.reshape(NW, JCH, JW)
    dispatch_sc, combine_sc = _sc_kernels()
    x_disp = dispatch_sc(x_te, slots3)
    out2 = _ffn_call(x_disp, gate_nfe, enc_nfe, dec_nef)
    out_te = combine_sc(out2, slots3)
    return out_te.reshape(b, l, e), loss[0, 0]


# speedup vs baseline: 1.0249x; 1.0058x over previous
import functools

import jax
import jax.numpy as jnp
from jax import lax
from jax.experimental import pallas as pl
from jax.experimental.pallas import tpu as pltpu
from jax.experimental.pallas import tpu_sc as plsc

EMBED = 1024
FF = 1024
NEXP = 64
CAP = 128
TOKENS = 8192
CHUNK = 512
NCHUNK = TOKENS // CHUNK
OVERFLOW = NEXP * CAP
EPG = 1
ROWS_PAD = OVERFLOW + EPG * CAP

NC, NS = 2, 16
NW = NC * NS
TPW = TOKENS // NW
JCH = 8
JW = TPW // JCH


def _router_body(x_ref, w_ref, b_ref, slots_ref, loss_ref, counts_ref, imp_ref):
    i = pl.program_id(0)

    @pl.when(i == 0)
    def _init():
        counts_ref[...] = jnp.zeros_like(counts_ref)
        imp_ref[...] = jnp.zeros_like(imp_ref)

    x = x_ref[...]
    w = w_ref[...]
    logits = lax.dot_general(x, w, (((1,), (1,)), ((), ())),
                             preferred_element_type=jnp.float32)
    logits = logits + b_ref[...]

    m = jnp.max(logits, axis=1, keepdims=True)
    p = jnp.exp(logits - m)
    scores = p / jnp.sum(p, axis=1, keepdims=True)
    imp_ref[...] += jnp.sum(scores, axis=0, keepdims=True)

    col = lax.broadcasted_iota(jnp.int32, (CHUNK, NEXP), 1)
    amax = jnp.min(jnp.where(logits == m, col, NEXP), axis=1)
    oh = (col == amax[:, None]).astype(jnp.float32)

    r_i = lax.broadcasted_iota(jnp.int32, (CHUNK, CHUNK), 0)
    c_i = lax.broadcasted_iota(jnp.int32, (CHUNK, CHUNK), 1)
    ltri = (r_i > c_i).astype(jnp.float32)
    within = lax.dot_general(ltri, oh, (((1,), (0,)), ((), ())),
                             preferred_element_type=jnp.float32)
    rank_f = (jnp.sum(within * oh, axis=1)
              + jnp.sum(oh * counts_ref[0, :][None, :], axis=1))
    counts_ref[...] += jnp.sum(oh, axis=0, keepdims=True)

    rank = rank_f.astype(jnp.int32)
    slot = jnp.where(rank < CAP, amax * CAP + rank, OVERFLOW)
    slots_ref[0, 0, :] = slot

    @pl.when(i == NCHUNK - 1)
    def _fin():
        counts = counts_ref[0, :]
        imp = imp_ref[0, :]
        loss = (NEXP / (float(TOKENS) * float(TOKENS))
                * jnp.sum(counts * imp))
        loss_ref[...] = jnp.broadcast_to(loss, loss_ref.shape)


_router_call = pl.pallas_call(
    _router_body,
    grid=(NCHUNK,),
    in_specs=[
        pl.BlockSpec((CHUNK, EMBED), lambda i: (i, 0)),
        pl.BlockSpec((NEXP, EMBED), lambda i: (0, 0)),
        pl.BlockSpec((1, NEXP), lambda i: (0, 0)),
    ],
    out_specs=[
        pl.BlockSpec((1, 1, CHUNK), lambda i: (i, 0, 0)),
        pl.BlockSpec((1, 128), lambda i: (0, 0)),
    ],
    out_shape=[
        jax.ShapeDtypeStruct((NCHUNK, 1, CHUNK), jnp.int32),
        jax.ShapeDtypeStruct((1, 128), jnp.float32),
    ],
    scratch_shapes=[
        pltpu.VMEM((1, NEXP), jnp.float32),
        pltpu.VMEM((1, NEXP), jnp.float32),
    ],
)


def _ffn_body(x_ref, g_ref, e_ref, d_ref, out_ref):
    i = pl.program_id(0)

    @pl.when(i < NEXP // EPG)
    def _compute():
        for k in range(EPG):
            x = x_ref[pl.ds(k * CAP, CAP), :].astype(jnp.bfloat16)
            g = g_ref[k].astype(jnp.bfloat16)
            en = e_ref[k].astype(jnp.bfloat16)
            de = d_ref[k].astype(jnp.bfloat16)
            h1 = lax.dot_general(x, g, (((1,), (1,)), ((), ())),
                                 preferred_element_type=jnp.float32)
            h2 = lax.dot_general(x, en, (((1,), (1,)), ((), ())),
                                 preferred_element_type=jnp.float32)
            act = (h1 * jax.nn.sigmoid(h1) * h2).astype(jnp.bfloat16)
            out_ref[pl.ds(k * CAP, CAP), :] = lax.dot_general(
                act, de, (((1,), (1,)), ((), ())),
                preferred_element_type=jnp.float32)

    @pl.when(i == NEXP // EPG)
    def _zero():
        out_ref[...] = jnp.zeros_like(out_ref)


_ffn_call = pl.pallas_call(
    _ffn_body,
    grid=(NEXP // EPG + 1,),
    in_specs=[
        pl.BlockSpec((EPG * CAP, EMBED), lambda i: (i, 0)),
        pl.BlockSpec((EPG, FF, EMBED),
                     lambda i: (jnp.minimum(i, NEXP // EPG - 1), 0, 0)),
        pl.BlockSpec((EPG, FF, EMBED),
                     lambda i: (jnp.minimum(i, NEXP // EPG - 1), 0, 0)),
        pl.BlockSpec((EPG, EMBED, FF),
                     lambda i: (jnp.minimum(i, NEXP // EPG - 1), 0, 0)),
    ],
    out_specs=pl.BlockSpec((EPG * CAP, EMBED), lambda i: (i, 0)),
    out_shape=jax.ShapeDtypeStruct((ROWS_PAD, EMBED), jnp.float32),
)


@functools.cache
def _sc_kernels():
    mesh = plsc.VectorSubcoreMesh(core_axis_name="c", subcore_axis_name="s",
                                  num_cores=NC, num_subcores=NS)
    scratch = [
        pltpu.VMEM((JCH, JW), jnp.int32),
        pltpu.VMEM((JW, EMBED), jnp.float32),
        pltpu.VMEM((JW, EMBED), jnp.float32),
        pltpu.SemaphoreType.DMA,
        pltpu.SemaphoreType.DMA,
        pltpu.SemaphoreType.DMA,
        pltpu.SemaphoreType.DMA,
    ]


    @functools.partial(
        pl.kernel,
        out_type=jax.ShapeDtypeStruct((ROWS_PAD, EMBED), jnp.float32),
        mesh=mesh, scratch_types=scratch,
    )
    def dispatch_sc(x_hbm, slots_hbm, xdisp_hbm, idx_v, rows0, rows1,
                    gsem0, gsem1, ssem0, ssem1):
        wid = lax.axis_index("s") * NC + lax.axis_index("c")
        bufs, gsems, ssems = (rows0, rows1), (gsem0, gsem1), (ssem0, ssem1)
        pltpu.sync_copy(slots_hbm.at[wid], idx_v)
        gets = [None] * JCH
        puts = [None] * JCH
        gets[0] = pltpu.async_copy(
            x_hbm.at[pl.ds(wid * TPW, JW)], bufs[0], gsems[0])
        for j in range(JCH):
            p = j % 2
            gets[j].wait()
            puts[j] = pltpu.async_copy(
                bufs[p], xdisp_hbm.at[idx_v.at[j]], ssems[p])
            if j + 1 < JCH:
                if j - 1 >= 0:
                    puts[j - 1].wait()
                gets[j + 1] = pltpu.async_copy(
                    x_hbm.at[pl.ds(wid * TPW + (j + 1) * JW, JW)],
                    bufs[(j + 1) % 2], gsems[(j + 1) % 2])
        puts[JCH - 2].wait()
        puts[JCH - 1].wait()

    @functools.partial(
        pl.kernel,
        out_type=jax.ShapeDtypeStruct((TOKENS, EMBED), jnp.float32),
        mesh=mesh, scratch_types=scratch,
    )
    def combine_sc(out2_hbm, slots_hbm, out_hbm, idx_v, rows0, rows1,
                   gsem0, gsem1, ssem0, ssem1):
        wid = lax.axis_index("s") * NC + lax.axis_index("c")
        bufs, gsems, ssems = (rows0, rows1), (gsem0, gsem1), (ssem0, ssem1)
        pltpu.sync_copy(slots_hbm.at[wid], idx_v)
        gets = [None] * JCH
        puts = [None] * JCH
        gets[0] = pltpu.async_copy(
            out2_hbm.at[idx_v.at[0]], bufs[0], gsems[0])
        for j in range(JCH):
            p = j % 2
            gets[j].wait()
            puts[j] = pltpu.async_copy(
                bufs[p], out_hbm.at[pl.ds(wid * TPW + j * JW, JW)], ssems[p])
            if j + 1 < JCH:
                if j - 1 >= 0:
                    puts[j - 1].wait()
                gets[j + 1] = pltpu.async_copy(
                    out2_hbm.at[idx_v.at[j + 1]],
                    bufs[(j + 1) % 2], gsems[(j + 1) % 2])
        puts[JCH - 2].wait()
        puts[JCH - 1].wait()

    return dispatch_sc, combine_sc


def kernel(x_ble, W_router, b_router, gate_nfe, enc_nfe, dec_nef):
    b, l, e = x_ble.shape
    x_te = x_ble.reshape(b * l, e)
    slots16, loss = _router_call(x_te, W_router, b_router.reshape(1, NEXP))
    slots3 = slots16.reshape(NW, JCH, JW)
    dispatch_sc, combine_sc = _sc_kernels()
    x_disp = dispatch_sc(x_te, slots3)
    out2 = _ffn_call(x_disp, gate_nfe, enc_nfe, dec_nef)
    out_te = combine_sc(out2, slots3)
    return out_te.reshape(b, l, e), loss[0, 0]
